# Initial kernel scaffold; baseline (speedup 1.0000x reference)
import functools

import jax
import jax.numpy as jnp
from jax import lax
from jax.experimental import pallas as pl
from jax.experimental.pallas import tpu as pltpu
from jax.experimental.pallas import tpu_sc as plsc

_NUM_USERS = 500000
_NUM_ITEMS = 500000
_EMBED_DIM = 64
_BATCH = 16384

_NC = 2
_NS = 16
_LANES = 16
_NW = _NC * _NS
_B_PER_W = _BATCH // _NW


def _sc_body(ids_hbm, user_hbm, item_hbm, out_hbm,
             ids_v, uidx_v, iidx_v, mu_v, mi_v, ubuf, ibuf, sem_u, sem_i):
    wid = lax.axis_index("s") * _NC + lax.axis_index("c")
    base = wid * _B_PER_W

    pltpu.sync_copy(ids_hbm.at[pl.ds(base, _B_PER_W)], ids_v)

    @pl.loop(0, _B_PER_W // _LANES, unroll=4)
    def _prep(g):
        off = g * _LANES
        ids = ids_v[pl.ds(off, _LANES)]
        is_user = (ids >= 1) & (ids <= _NUM_USERS)
        is_item = ids > _NUM_USERS
        uidx = jnp.clip(ids - 1, 0, _NUM_USERS - 1)
        iidx = jnp.clip(ids - (_NUM_USERS + 1), 0, _NUM_ITEMS - 1)
        uidx_v[pl.ds(off, _LANES)] = uidx
        iidx_v[pl.ds(off, _LANES)] = iidx
        mu_v[pl.ds(off, _LANES)] = jnp.where(is_user, 1.0, 0.0).astype(jnp.float32)
        mi_v[pl.ds(off, _LANES)] = jnp.where(is_item, 1.0, 0.0).astype(jnp.float32)

    cp_u = pltpu.async_copy(user_hbm.at[uidx_v], ubuf, sem_u)
    cp_i = pltpu.async_copy(item_hbm.at[iidx_v], ibuf, sem_i)
    cp_u.wait()
    cp_i.wait()

    @pl.loop(0, _B_PER_W, unroll=4)
    def _blend(r):
        mu = jnp.full((_LANES,), mu_v[r], dtype=jnp.float32)
        mi = jnp.full((_LANES,), mi_v[r], dtype=jnp.float32)
        for c in range(_EMBED_DIM // _LANES):
            sl = pl.ds(c * _LANES, _LANES)
            ubuf[r, sl] = ubuf[r, sl] * mu + ibuf[r, sl] * mi

    pltpu.sync_copy(ubuf, out_hbm.at[pl.ds(base, _B_PER_W)])


@jax.jit
def _hybrid_features(ids32, user_emb, item_emb):
    mesh = plsc.VectorSubcoreMesh(
        core_axis_name="c", subcore_axis_name="s",
        num_cores=_NC, num_subcores=_NS)
    return pl.kernel(
        _sc_body,
        out_type=jax.ShapeDtypeStruct((_BATCH, _EMBED_DIM), jnp.float32),
        mesh=mesh,
        scratch_types=[
            pltpu.VMEM((_B_PER_W,), jnp.int32),
            pltpu.VMEM((_B_PER_W,), jnp.int32),
            pltpu.VMEM((_B_PER_W,), jnp.int32),
            pltpu.VMEM((_B_PER_W,), jnp.float32),
            pltpu.VMEM((_B_PER_W,), jnp.float32),
            pltpu.VMEM((_B_PER_W, _EMBED_DIM), jnp.float32),
            pltpu.VMEM((_B_PER_W, _EMBED_DIM), jnp.float32),
            pltpu.SemaphoreType.DMA,
            pltpu.SemaphoreType.DMA,
        ],
    )(ids32, user_emb, item_emb)


def kernel(node_ids, user_emb, item_emb):
    ids32 = node_ids.astype(jnp.int32)
    return _hybrid_features(ids32, user_emb, item_emb)


# speedup vs baseline: 1.2277x; 1.2277x over previous
import functools

import jax
import jax.numpy as jnp
from jax import lax
from jax.experimental import pallas as pl
from jax.experimental.pallas import tpu as pltpu
from jax.experimental.pallas import tpu_sc as plsc

_NUM_USERS = 500000
_NUM_ITEMS = 500000
_EMBED_DIM = 64
_BATCH = 16384

_NC = 2
_NS = 16
_LANES = 16
_NW = _NC * _NS
_B_PER_W = _BATCH // _NW


def _sc_body(ids_hbm, user_hbm, item_hbm, out_hbm,
             ids_v, uidx_v, iidx_v, mu_v, mi_v, ubuf, ibuf, sem_u, sem_i):
    wid = lax.axis_index("s") * _NC + lax.axis_index("c")
    base = wid * _B_PER_W

    pltpu.sync_copy(ids_hbm.at[pl.ds(base, _B_PER_W)], ids_v)

    @pl.loop(0, _B_PER_W // _LANES, unroll=4)
    def _prep(g):
        off = g * _LANES
        ids = ids_v[pl.ds(off, _LANES)]
        is_user = (ids >= 1) & (ids <= _NUM_USERS)
        is_item = ids > _NUM_USERS
        uidx = jnp.clip(ids - 1, 0, _NUM_USERS - 1)
        iidx = jnp.clip(ids - (_NUM_USERS + 1), 0, _NUM_ITEMS - 1)
        uidx_v[pl.ds(off, _LANES)] = uidx
        iidx_v[pl.ds(off, _LANES)] = iidx
        mu_v[pl.ds(off, _LANES)] = jnp.where(is_user, 1.0, 0.0).astype(jnp.float32)
        mi_v[pl.ds(off, _LANES)] = jnp.where(is_item, 1.0, 0.0).astype(jnp.float32)

    cp_u = pltpu.async_copy(user_hbm.at[uidx_v], ubuf, sem_u)
    cp_i = pltpu.async_copy(item_hbm.at[iidx_v], ibuf, sem_i)
    cp_u.wait()
    cp_i.wait()

    @pl.loop(0, _B_PER_W // _LANES)
    def _blend(g):
        off = g * _LANES
        mu16 = mu_v[pl.ds(off, _LANES)]
        mi16 = mi_v[pl.ds(off, _LANES)]
        for j in range(_LANES):
            r = off + j
            mu = jnp.broadcast_to(mu16[j], (_LANES,))
            mi = jnp.broadcast_to(mi16[j], (_LANES,))
            for c in range(_EMBED_DIM // _LANES):
                sl = pl.ds(c * _LANES, _LANES)
                ubuf[r, sl] = ubuf[r, sl] * mu + ibuf[r, sl] * mi

    pltpu.sync_copy(ubuf, out_hbm.at[pl.ds(base, _B_PER_W)])


@jax.jit
def _hybrid_features(ids32, user_emb, item_emb):
    mesh = plsc.VectorSubcoreMesh(
        core_axis_name="c", subcore_axis_name="s",
        num_cores=_NC, num_subcores=_NS)
    return pl.kernel(
        _sc_body,
        out_type=jax.ShapeDtypeStruct((_BATCH, _EMBED_DIM), jnp.float32),
        mesh=mesh,
        compiler_params=pltpu.CompilerParams(use_tc_tiling_on_sc=False),
        scratch_types=[
            pltpu.VMEM((_B_PER_W,), jnp.int32),
            pltpu.VMEM((_B_PER_W,), jnp.int32),
            pltpu.VMEM((_B_PER_W,), jnp.int32),
            pltpu.VMEM((_B_PER_W,), jnp.float32),
            pltpu.VMEM((_B_PER_W,), jnp.float32),
            pltpu.VMEM((_B_PER_W, _EMBED_DIM), jnp.float32),
            pltpu.VMEM((_B_PER_W, _EMBED_DIM), jnp.float32),
            pltpu.SemaphoreType.DMA,
            pltpu.SemaphoreType.DMA,
        ],
    )(ids32, user_emb, item_emb)


def kernel(node_ids, user_emb, item_emb):
    ids32 = node_ids.astype(jnp.int32)
    return _hybrid_features(ids32, user_emb, item_emb)


# speedup vs baseline: 2.1932x; 1.7864x over previous
import jax
import jax.numpy as jnp
from jax import lax
from jax.experimental import pallas as pl
from jax.experimental.pallas import tpu as pltpu
from jax.experimental.pallas import tpu_sc as plsc

_NUM_USERS = 500000
_NUM_ITEMS = 500000
_EMBED_DIM = 64
_BATCH = 16384

_NC = 2
_NS = 16
_LANES = 16
_NW = _NC * _NS
_B_PER_W = _BATCH // _NW
_GROUPS = _B_PER_W // _LANES


def _sc_body(ids_hbm, user_hbm, item_hbm, out_hbm, ids_v, stage, outbuf, sem):
    wid = lax.axis_index("s") * _NC + lax.axis_index("c")
    base = wid * _B_PER_W

    pltpu.sync_copy(ids_hbm.at[pl.ds(base, _B_PER_W)], ids_v)

    @pl.loop(0, _GROUPS)
    def _group(g):
        off = g * _LANES
        ids = ids_v[pl.ds(off, _LANES)]
        is_item = ids > _NUM_USERS
        is_pad = ids == 0
        urow = jnp.clip(ids - 1, 0, _NUM_USERS - 1)
        irow = jnp.clip(ids - (_NUM_USERS + 1), 0, _NUM_ITEMS - 1)
        row = jnp.where(is_item, irow, urow)
        brow = row & jnp.int32(~7)
        sub = row & jnp.int32(7)
        mval = jnp.where(is_pad, jnp.float32(0.0), jnp.float32(1.0))
        item_sel = jnp.where(is_item, jnp.int32(1), jnp.int32(0))

        for j in range(_LANES):
            b = pl.multiple_of(brow[j], 8)
            sel = item_sel[j]

            @pl.when(sel != 0)
            def _():
                pltpu.async_copy(item_hbm.at[pl.ds(b, 8), :], stage.at[j], sem)

            @pl.when(sel == 0)
            def _():
                pltpu.async_copy(user_hbm.at[pl.ds(b, 8), :], stage.at[j], sem)

        for j in range(_LANES):
            pltpu.make_async_copy(user_hbm.at[pl.ds(0, 8), :], stage.at[j], sem).wait()

        for j in range(_LANES):
            s = sub[j]
            m = jnp.broadcast_to(mval[j], (_LANES,))
            r = off + j
            for c in range(_EMBED_DIM // _LANES):
                sl = pl.ds(c * _LANES, _LANES)
                outbuf[r, sl] = stage[j, s, sl] * m

    pltpu.sync_copy(outbuf, out_hbm.at[pl.ds(base, _B_PER_W)])


@jax.jit
def _hybrid_features(ids32, user_emb, item_emb):
    mesh = plsc.VectorSubcoreMesh(
        core_axis_name="c", subcore_axis_name="s",
        num_cores=_NC, num_subcores=_NS)
    return pl.kernel(
        _sc_body,
        out_type=jax.ShapeDtypeStruct((_BATCH, _EMBED_DIM), jnp.float32),
        mesh=mesh,
        scratch_types=[
            pltpu.VMEM((_B_PER_W,), jnp.int32),
            pltpu.VMEM((_LANES, 8, _EMBED_DIM), jnp.float32),
            pltpu.VMEM((_B_PER_W, _EMBED_DIM), jnp.float32),
            pltpu.SemaphoreType.DMA,
        ],
    )(ids32, user_emb, item_emb)


def kernel(node_ids, user_emb, item_emb):
    ids32 = node_ids.astype(jnp.int32)
    return _hybrid_features(ids32, user_emb, item_emb)
